# baseline (speedup 1.0000x reference)
import jax
import jax.numpy as jnp
from jax import lax
from jax.experimental import pallas as pl
from jax.experimental.pallas import tpu as pltpu
from jax.experimental.pallas import tpu_sc as plsc

_NC = 2
_NS = 16
_NW = _NC * _NS
_LANES = 16

_B = 1024
_C = 20
_NNEG = 20
_DIM = 64
_VOCAB = 100000
_CA = _C + _C * _NNEG
_CHUNK = 112
_NCHUNK = 4
_CP = _CHUNK * _NCHUNK
_BPW = _B // _NW
_HB = _CP // 2
_NH = _BPW * 2
_GPH = _HB // _LANES
_W2 = 2 * _DIM


def _sc_scores_body(emb_i_hbm, emb_o2_hbm, iword_hbm, cidx2_hbm, par_hbm,
                    scores_hbm,
                    iw_v, ivecs_v, idx_v, par_v, rows_v0, rows_v1, scores_v,
                    tba_v, tbb_v, sem_i, sem0, sem1):
    wid = lax.axis_index("s") * _NC + lax.axis_index("c")
    base = wid * _BPW

    pltpu.sync_copy(iword_hbm.at[pl.ds(base, _BPW)], iw_v)
    pltpu.async_copy(emb_i_hbm.at[iw_v], ivecs_v, sem_i).wait()
    pltpu.sync_copy(cidx2_hbm.at[pl.ds(base * _NCHUNK, _BPW * _NCHUNK)], idx_v)
    pltpu.sync_copy(par_hbm.at[pl.ds(base * _CP, _BPW * _CP)], par_v)

    rows_bufs = (rows_v0, rows_v1)
    sems = (sem0, sem1)

    def fire(h, buf, sem):
        for k in range(2):
            pltpu.async_copy(
                emb_o2_hbm.at[idx_v.at[h * 2 + k]],
                buf.at[pl.ds(k * _CHUNK, _CHUNK)],
                sem,
            )

    def drain(h, buf, sem):
        for k in range(2):
            pltpu.make_async_copy(
                emb_o2_hbm.at[idx_v.at[h * 2 + k]],
                buf.at[pl.ds(k * _CHUNK, _CHUNK)],
                sem,
            ).wait()

    iota = lax.iota(jnp.int32, _LANES)
    tcols = [(iota * 0 + l, iota) for l in range(_LANES)]

    def compute_h(h, rows):
        b = h // 2
        sbase = h * _HB
        iv = [ivecs_v[b, pl.ds(k * _LANES, _LANES)] for k in range(4)]

        def group(g, _):
            jbase = g * _LANES
            for r in range(_LANES):
                j = jbase + r
                va = rows[j, pl.ds(0, _LANES)] * iv[0]
                va = va + rows[j, pl.ds(_LANES, _LANES)] * iv[1]
                va = va + rows[j, pl.ds(2 * _LANES, _LANES)] * iv[2]
                va = va + rows[j, pl.ds(3 * _LANES, _LANES)] * iv[3]
                tba_v[r, pl.ds(0, _LANES)] = va
                vb = rows[j, pl.ds(4 * _LANES, _LANES)] * iv[0]
                vb = vb + rows[j, pl.ds(5 * _LANES, _LANES)] * iv[1]
                vb = vb + rows[j, pl.ds(6 * _LANES, _LANES)] * iv[2]
                vb = vb + rows[j, pl.ds(7 * _LANES, _LANES)] * iv[3]
                tbb_v[r, pl.ds(0, _LANES)] = vb
            sa = plsc.load_gather(tba_v, [tcols[0][1], tcols[0][0]])
            sb = plsc.load_gather(tbb_v, [tcols[0][1], tcols[0][0]])
            for l in range(1, _LANES):
                sa = sa + plsc.load_gather(tba_v, [tcols[l][1], tcols[l][0]])
                sb = sb + plsc.load_gather(tbb_v, [tcols[l][1], tcols[l][0]])
            pvec = par_v[pl.ds(sbase + jbase, _LANES)]
            svec = jnp.where(pvec > 0, sb, sa)
            scores_v[pl.ds(sbase + jbase, _LANES)] = svec
            return 0

        lax.fori_loop(0, _GPH, group, 0)

    fire(0, rows_bufs[0], sems[0])
    fire(1, rows_bufs[1], sems[1])

    def pair(i, _):
        h0 = i * 2
        for p in range(2):
            h = h0 + p
            drain(h, rows_bufs[p], sems[p])
            compute_h(h, rows_bufs[p])

            @pl.when(h + 2 < _NH)
            def _():
                fire(h + 2, rows_bufs[p], sems[p])

        return 0

    lax.fori_loop(0, _NH // 2, pair, 0)

    pltpu.sync_copy(scores_v, scores_hbm.at[pl.ds(base * _CP, _BPW * _CP)])


def _sc_scores(emb_i, emb_o2, iword, cidx2, par):
    mesh = plsc.VectorSubcoreMesh(core_axis_name="c", subcore_axis_name="s")
    return pl.kernel(
        _sc_scores_body,
        out_type=jax.ShapeDtypeStruct((_B * _CP,), jnp.float32),
        mesh=mesh,
        compiler_params=pltpu.CompilerParams(
            needs_layout_passes=False, use_tc_tiling_on_sc=False
        ),
        scratch_types=[
            pltpu.VMEM((_BPW,), jnp.int32),
            pltpu.VMEM((_BPW, _DIM), jnp.float32),
            pltpu.VMEM((_BPW * _NCHUNK, _CHUNK), jnp.int32),
            pltpu.VMEM((_BPW * _CP,), jnp.int32),
            pltpu.VMEM((_HB, _W2), jnp.float32),
            pltpu.VMEM((_HB, _W2), jnp.float32),
            pltpu.VMEM((_BPW * _CP,), jnp.float32),
            pltpu.VMEM((_LANES, _LANES), jnp.float32),
            pltpu.VMEM((_LANES, _LANES), jnp.float32),
            pltpu.SemaphoreType.DMA,
            pltpu.SemaphoreType.DMA,
            pltpu.SemaphoreType.DMA,
        ],
    )(emb_i, emb_o2, iword, cidx2, par)


def _tc_loss_body(s_ref, o_ref):
    s = s_ref[...]
    col = lax.broadcasted_iota(jnp.int32, (_B, _CP), 1)
    x = jnp.where(col < _C, s, -s)
    ls = jnp.minimum(x, 0.0) - jnp.log(1.0 + jnp.exp(-jnp.abs(x)))
    ls = jnp.where(col < _CA, ls, 0.0)
    o_ref[0, 0] = -jnp.sum(ls) / (_B * _C)


def _tc_loss(scores):
    return pl.pallas_call(
        _tc_loss_body,
        out_shape=jax.ShapeDtypeStruct((1, 1), jnp.float32),
        in_specs=[pl.BlockSpec(memory_space=pltpu.VMEM)],
        out_specs=pl.BlockSpec(memory_space=pltpu.SMEM),
    )(scores)


def kernel(iword, owords, nwords, emb_i, emb_o):
    iw = iword.astype(jnp.int32)
    pad = jnp.zeros((_B, _CP - _CA), jnp.int32)
    cidx = jnp.concatenate(
        [owords.astype(jnp.int32), nwords.astype(jnp.int32), pad], axis=1
    )
    cidx2 = (cidx // 2).reshape(_B * _NCHUNK, _CHUNK)
    par = (cidx & 1).reshape(_B * _CP)
    emb_o2 = emb_o.reshape(_VOCAB // 2, _W2)
    scores = _sc_scores(emb_i, emb_o2, iw, cidx2, par)
    loss = _tc_loss(scores.reshape(_B, _CP))
    return jnp.reshape(loss, ())
